# baseline (speedup 1.0000x reference)
import functools

import jax
import jax.numpy as jnp
from jax import lax
from jax.experimental import pallas as pl
from jax.experimental.pallas import tpu as pltpu
from jax.experimental.pallas import tpu_sc as plsc

VOCAB = 8192
NC = 2
NS = 16
NW = NC * NS
B = 8192
BPW = B // NW
C = 4
NCH = BPW // C


def _sc_gather(idx_r, table):
    mesh = plsc.VectorSubcoreMesh(core_axis_name="c", subcore_axis_name="s")

    @functools.partial(
        pl.kernel,
        mesh=mesh,
        out_type=jax.ShapeDtypeStruct((B, VOCAB), jnp.float32),
        scratch_types=[
            pltpu.VMEM((NCH, C), jnp.int32),
            pltpu.VMEM((C, VOCAB), jnp.float32),
            pltpu.VMEM((C, VOCAB), jnp.float32),
            pltpu.SemaphoreType.DMA,
        ],
    )
    def k(idx_hbm, table_hbm, out_hbm, idx_v, buf0, buf1, gsem):
        wid = lax.axis_index("s") * NC + lax.axis_index("c")
        base = wid * BPW
        pltpu.sync_copy(idx_hbm.at[wid], idx_v)

        def body(ci, carry):
            pltpu.async_copy(table_hbm.at[idx_v.at[ci]], buf0, gsem)
            pltpu.async_copy(buf1, out_hbm.at[pl.ds(base + ci * C, C)], gsem)
            return carry

        lax.fori_loop(0, NCH, body, 0, unroll=False)

        def drain(ci, carry):
            pltpu.make_async_copy(table_hbm.at[idx_v.at[0]], buf0, gsem).wait()
            pltpu.make_async_copy(buf1, out_hbm.at[pl.ds(base, C)], gsem).wait()
            return carry

        lax.fori_loop(0, NCH, drain, 0, unroll=False)

    return k(idx_r, table)


def kernel(idx, table):
    idx_r = idx.reshape(NW, NCH, C).astype(jnp.int32)
    return _sc_gather(idx_r, table)


# speedup vs baseline: 1.0022x; 1.0022x over previous
import functools

import jax
import jax.numpy as jnp
from jax import lax
from jax.experimental import pallas as pl
from jax.experimental.pallas import tpu as pltpu
from jax.experimental.pallas import tpu_sc as plsc

VOCAB = 8192
NC = 2
NS = 16
NW = NC * NS
B = 8192
BPW = B // NW
C = 2
NCH = BPW // C
NBUF = 4
GROUPS = NCH // NBUF


def _sc_gather(idx_r, table):
    mesh = plsc.VectorSubcoreMesh(core_axis_name="c", subcore_axis_name="s")

    @functools.partial(
        pl.kernel,
        mesh=mesh,
        out_type=jax.ShapeDtypeStruct((B, VOCAB), jnp.float32),
        scratch_types=[
            pltpu.VMEM((NCH, C), jnp.int32),
            pltpu.VMEM((NBUF, C, VOCAB), jnp.float32),
            pltpu.SemaphoreType.DMA((NBUF,)),
            pltpu.SemaphoreType.DMA((NBUF,)),
        ],
    )
    def k(idx_hbm, table_hbm, out_hbm, idx_v, bufs, gsem, osem):
        wid = lax.axis_index("s") * NC + lax.axis_index("c")
        base = wid * BPW
        pltpu.sync_copy(idx_hbm.at[wid], idx_v)

        def gather(ci, b):
            pltpu.async_copy(table_hbm.at[idx_v.at[ci]], bufs.at[b], gsem.at[b])

        def wait_gather(ci, b):
            pltpu.make_async_copy(
                table_hbm.at[idx_v.at[ci]], bufs.at[b], gsem.at[b]
            ).wait()

        def writeback(ci, b):
            pltpu.async_copy(
                bufs.at[b], out_hbm.at[pl.ds(base + ci * C, C)], osem.at[b]
            )

        def wait_writeback(b):
            pltpu.make_async_copy(
                bufs.at[b], out_hbm.at[pl.ds(base, C)], osem.at[b]
            ).wait()

        for b in range(NBUF - 1):
            gather(b, b)

        def body(j, carry):
            ci0 = j * NBUF
            for b in range(NBUF):
                ci = ci0 + b
                bn = (b + NBUF - 1) % NBUF
                wait_gather(ci, b)

                @pl.when(ci >= 1)
                def _():
                    wait_writeback(bn)

                @pl.when(ci + NBUF - 1 < NCH)
                def _():
                    gather(ci + NBUF - 1, bn)

                writeback(ci, b)
            return carry

        lax.fori_loop(0, GROUPS, body, 0, unroll=False)
        wait_writeback((NCH - 1) % NBUF)

    return k(idx_r, table)


def kernel(idx, table):
    idx_r = idx.reshape(NW, NCH, C).astype(jnp.int32)
    return _sc_gather(idx_r, table)
